# baseline (speedup 1.0000x reference)
import jax
import jax.numpy as jnp
from jax import lax
from jax.experimental import pallas as pl
from jax.experimental.pallas import tpu as pltpu
from jax.experimental.pallas import tpu_sc as plsc

N = 10000
NP = 10240
E = 320000
D = 128
NC = 2
NS = 16
NW = NC * NS
CH = 128
KCH = 80
EPW = CH * KCH
EP = NW * EPW
NB = 4
RPT = NP // NS
ZF = RPT // CH

_MESH = plsc.VectorSubcoreMesh(core_axis_name="c", subcore_axis_name="s")


def _fill_rows(buf, nrow, ncol, vec):
    def body(i, _):
        for j in range(ncol // 16):
            buf[i, pl.ds(j * 16, 16)] = vec
        return 0

    lax.fori_loop(0, nrow, body, 0)


def _cnt_body(dst_hbm, out_hbm, di, ones_v, acc, i0, i1):
    c = lax.axis_index("c")
    s = lax.axis_index("s")
    w = c * NS + s
    isem = (i0, i1)
    _fill_rows(ones_v, CH, D, jnp.zeros((16,), jnp.float32))
    for k in range(ZF):
        pltpu.sync_copy(ones_v, acc.at[pl.ds(s * RPT + k * CH, CH)])
    plsc.subcore_barrier()
    _fill_rows(ones_v, CH, D, jnp.ones((16,), jnp.float32))
    base = w * EPW
    for j in range(2):
        pltpu.async_copy(dst_hbm.at[pl.ds(base + j * CH, CH)], di.at[j],
                         isem[j])

    def chunk(t, _):
        for slot in range(2):
            ci = t * 2 + slot
            pltpu.make_async_copy(dst_hbm.at[pl.ds(base + ci * CH, CH)],
                                  di.at[slot], isem[slot]).wait()
            pltpu.sync_copy(ones_v, acc.at[di.at[slot]], add=True)
            nl = ci + 2

            @pl.when(nl < KCH)
            def _():
                pltpu.async_copy(dst_hbm.at[pl.ds(base + nl * CH, CH)],
                                 di.at[slot], isem[slot])

        return 0

    lax.fori_loop(0, KCH // 2, chunk, 0)
    plsc.subcore_barrier()
    for k in range(ZF):
        pltpu.sync_copy(acc.at[pl.ds(s * RPT + k * CH, CH)],
                        out_hbm.at[pl.ds(c * NP + s * RPT + k * CH, CH)])


_cnt_call = pl.kernel(
    _cnt_body,
    out_type=jax.ShapeDtypeStruct((2 * NP, D), jnp.float32),
    mesh=_MESH,
    scratch_types=[
        pltpu.VMEM((2, CH), jnp.int32),
        pltpu.VMEM((CH, D), jnp.float32),
        pltpu.VMEM_SHARED((NP, D), jnp.float32),
        pltpu.SemaphoreType.DMA,
        pltpu.SemaphoreType.DMA,
    ],
)


def _agg_body(hs_hbm, src_hbm, dst_hbm, out_hbm,
              si, di, r0, r1, acc, i0, i1, i2, i3, g0, g1):
    c = lax.axis_index("c")
    s = lax.axis_index("s")
    w = c * NS + s
    rows = (r0, r1)
    gsem = (g0, g1)
    isem = (i0, i1, i2, i3)
    _fill_rows(r0, CH, D, jnp.zeros((16,), jnp.float32))
    for k in range(ZF):
        pltpu.sync_copy(r0, acc.at[pl.ds(s * RPT + k * CH, CH)])
    plsc.subcore_barrier()
    base = w * EPW
    for j in range(4):
        pltpu.async_copy(src_hbm.at[pl.ds(base + j * CH, CH)], si.at[j],
                         isem[j])
        pltpu.async_copy(dst_hbm.at[pl.ds(base + j * CH, CH)], di.at[j],
                         isem[j])
    for b in range(2):
        pltpu.make_async_copy(src_hbm.at[pl.ds(base + b * CH, CH)],
                              si.at[b], isem[b]).wait()
        pltpu.make_async_copy(dst_hbm.at[pl.ds(base + b * CH, CH)],
                              di.at[b], isem[b]).wait()
        pltpu.async_copy(hs_hbm.at[si.at[b]], rows[b], gsem[b])

    def outer(t, _):
        for j in range(4):
            ci = t * 4 + j
            b = j % 2
            slot = j
            pltpu.make_async_copy(hs_hbm.at[si.at[slot]], rows[b],
                                  gsem[b]).wait()
            pltpu.sync_copy(rows[b], acc.at[di.at[slot]], add=True)
            nl = ci + 4

            @pl.when(nl < KCH)
            def _():
                pltpu.async_copy(src_hbm.at[pl.ds(base + nl * CH, CH)],
                                 si.at[slot], isem[slot])
                pltpu.async_copy(dst_hbm.at[pl.ds(base + nl * CH, CH)],
                                 di.at[slot], isem[slot])

            ng = ci + 2
            slot2 = (j + 2) % 4

            @pl.when(ng < KCH)
            def _():
                pltpu.make_async_copy(
                    src_hbm.at[pl.ds(base + ng * CH, CH)], si.at[slot2],
                    isem[slot2]).wait()
                pltpu.make_async_copy(
                    dst_hbm.at[pl.ds(base + ng * CH, CH)], di.at[slot2],
                    isem[slot2]).wait()
                pltpu.async_copy(hs_hbm.at[si.at[slot2]], rows[b], gsem[b])

        return 0

    lax.fori_loop(0, KCH // 4, outer, 0)
    plsc.subcore_barrier()
    for k in range(ZF):
        pltpu.sync_copy(acc.at[pl.ds(s * RPT + k * CH, CH)],
                        out_hbm.at[pl.ds(c * NP + s * RPT + k * CH, CH)])


_agg_call = pl.kernel(
    _agg_body,
    out_type=jax.ShapeDtypeStruct((2 * NP, D), jnp.float32),
    mesh=_MESH,
    scratch_types=[
        pltpu.VMEM((4, CH), jnp.int32),
        pltpu.VMEM((4, CH), jnp.int32),
        pltpu.VMEM((CH, D), jnp.float32),
        pltpu.VMEM((CH, D), jnp.float32),
        pltpu.VMEM_SHARED((NP, D), jnp.float32),
        pltpu.SemaphoreType.DMA,
        pltpu.SemaphoreType.DMA,
        pltpu.SemaphoreType.DMA,
        pltpu.SemaphoreType.DMA,
        pltpu.SemaphoreType.DMA,
        pltpu.SemaphoreType.DMA,
    ],
)

BR = 1024


def _pre_body(cnt_ref, x_ref, w_ref, dinv_ref, hs_ref):
    deg = cnt_ref[0] + cnt_ref[1] + 1.0
    dinvb = lax.rsqrt(jnp.maximum(deg, 1e-12))
    h = jnp.dot(x_ref[...], w_ref[...], preferred_element_type=jnp.float32)
    dinv_ref[...] = dinvb
    hs_ref[...] = h * dinvb


_pre_call = pl.pallas_call(
    _pre_body,
    grid=(pl.cdiv(N, BR),),
    in_specs=[
        pl.BlockSpec((2, BR, D), lambda i: (0, i, 0)),
        pl.BlockSpec((BR, D), lambda i: (i, 0)),
        pl.BlockSpec((D, D), lambda i: (0, 0)),
    ],
    out_specs=[
        pl.BlockSpec((BR, D), lambda i: (i, 0)),
        pl.BlockSpec((BR, D), lambda i: (i, 0)),
    ],
    out_shape=[
        jax.ShapeDtypeStruct((N, D), jnp.float32),
        jax.ShapeDtypeStruct((N, D), jnp.float32),
    ],
)


def _mid_body(p_ref, hs1_ref, dinv_ref, b1_ref, w2_ref, hs2_ref):
    agg = p_ref[0] + p_ref[1] + hs1_ref[...]
    t = agg * dinv_ref[...] + b1_ref[...]
    t = jnp.maximum(t, 0.0)
    h2 = jnp.dot(t, w2_ref[...], preferred_element_type=jnp.float32)
    hs2_ref[...] = h2 * dinv_ref[...]


_mid_call = pl.pallas_call(
    _mid_body,
    grid=(pl.cdiv(N, BR),),
    in_specs=[
        pl.BlockSpec((2, BR, D), lambda i: (0, i, 0)),
        pl.BlockSpec((BR, D), lambda i: (i, 0)),
        pl.BlockSpec((BR, D), lambda i: (i, 0)),
        pl.BlockSpec((1, D), lambda i: (0, 0)),
        pl.BlockSpec((D, D), lambda i: (0, 0)),
    ],
    out_specs=pl.BlockSpec((BR, D), lambda i: (i, 0)),
    out_shape=jax.ShapeDtypeStruct((N, D), jnp.float32),
)


def _fin_body(q_ref, hs2_ref, dinv_ref, b2_ref, out_ref):
    agg = q_ref[0] + q_ref[1] + hs2_ref[...]
    out_ref[...] = agg * dinv_ref[...] + b2_ref[...]


_fin_call = pl.pallas_call(
    _fin_body,
    grid=(pl.cdiv(N, BR),),
    in_specs=[
        pl.BlockSpec((2, BR, D), lambda i: (0, i, 0)),
        pl.BlockSpec((BR, D), lambda i: (i, 0)),
        pl.BlockSpec((BR, D), lambda i: (i, 0)),
        pl.BlockSpec((1, D), lambda i: (0, 0)),
    ],
    out_specs=pl.BlockSpec((BR, D), lambda i: (i, 0)),
    out_shape=jax.ShapeDtypeStruct((N, D), jnp.float32),
)


def kernel(x, edge_index, W1, b1, W2, b2):
    src = edge_index[0]
    dst = edge_index[1]
    pad = EP - E
    srcp = jnp.concatenate([src, jnp.zeros((pad,), src.dtype)])
    dstp = jnp.concatenate([dst, jnp.full((pad,), N, dst.dtype)])
    cntp = _cnt_call(dstp).reshape(2, NP, D)
    dinvb, hs1 = _pre_call(cntp, x, W1)
    p = _agg_call(hs1, srcp, dstp).reshape(2, NP, D)
    hs2 = _mid_call(p, hs1, dinvb, b1.reshape(1, D), W2)
    q = _agg_call(hs2, srcp, dstp).reshape(2, NP, D)
    return _fin_call(q, hs2, dinvb, b2.reshape(1, D))


# speedup vs baseline: 3.3614x; 3.3614x over previous
import jax
import jax.numpy as jnp
from jax import lax
from jax.experimental import pallas as pl
from jax.experimental.pallas import tpu as pltpu
from jax.experimental.pallas import tpu_sc as plsc

N = 10000
NP = 10240
E = 320000
D = 128
NC = 2
NS = 16
NW = NC * NS
CH = 128
KCH = 80
EPW = CH * KCH
EP = NW * EPW
NB = 4
RPT = NP // NS
ZF = RPT // CH

_MESH = plsc.VectorSubcoreMesh(core_axis_name="c", subcore_axis_name="s")


def _fill_rows(buf, nrow, ncol, vec):
    def body(i, _):
        for j in range(ncol // 16):
            buf[i, pl.ds(j * 16, 16)] = vec
        return 0

    lax.fori_loop(0, nrow, body, 0)


def _cnt_body(dst_hbm, out_hbm, di, ones_v, acc, i0, i1):
    c = lax.axis_index("c")
    s = lax.axis_index("s")
    w = c * NS + s
    isem = (i0, i1)
    _fill_rows(ones_v, CH, D, jnp.zeros((16,), jnp.float32))
    for k in range(ZF):
        pltpu.sync_copy(ones_v, acc.at[pl.ds(s * RPT + k * CH, CH)])
    plsc.subcore_barrier()
    _fill_rows(ones_v, CH, D, jnp.ones((16,), jnp.float32))
    base = w * EPW
    for j in range(2):
        pltpu.async_copy(dst_hbm.at[pl.ds(base + j * CH, CH)], di.at[j],
                         isem[j])

    def chunk(t, _):
        for slot in range(2):
            ci = t * 2 + slot
            pltpu.make_async_copy(dst_hbm.at[pl.ds(base + ci * CH, CH)],
                                  di.at[slot], isem[slot]).wait()
            pltpu.sync_copy(ones_v, acc.at[di.at[slot]], add=True)
            nl = ci + 2

            @pl.when(nl < KCH)
            def _():
                pltpu.async_copy(dst_hbm.at[pl.ds(base + nl * CH, CH)],
                                 di.at[slot], isem[slot])

        return 0

    lax.fori_loop(0, KCH // 2, chunk, 0)
    plsc.subcore_barrier()
    for k in range(ZF):
        pltpu.sync_copy(acc.at[pl.ds(s * RPT + k * CH, CH)],
                        out_hbm.at[pl.ds(c * NP + s * RPT + k * CH, CH)])


_cnt_call = pl.kernel(
    _cnt_body,
    out_type=jax.ShapeDtypeStruct((2 * NP, D), jnp.float32),
    mesh=_MESH,
    scratch_types=[
        pltpu.VMEM((2, CH), jnp.int32),
        pltpu.VMEM((CH, D), jnp.float32),
        pltpu.VMEM_SHARED((NP, D), jnp.float32),
        pltpu.SemaphoreType.DMA,
        pltpu.SemaphoreType.DMA,
    ],
)


def _agg_body(hs_hbm, src_hbm, dst_hbm, out_hbm,
              si, di, r0, r1, acc, i0, i1, i2, i3, g0, g1):
    c = lax.axis_index("c")
    s = lax.axis_index("s")
    w = c * NS + s
    rows = (r0, r1)
    gsem = (g0, g1)
    isem = (i0, i1, i2, i3)
    _fill_rows(r0, CH, D, jnp.zeros((16,), jnp.float32))
    for k in range(ZF):
        pltpu.sync_copy(r0, acc.at[pl.ds(s * RPT + k * CH, CH)])
    plsc.subcore_barrier()
    base = w * EPW
    for j in range(4):
        pltpu.async_copy(src_hbm.at[pl.ds(base + j * CH, CH)], si.at[j],
                         isem[j])
        pltpu.async_copy(dst_hbm.at[pl.ds(base + j * CH, CH)], di.at[j],
                         isem[j])
    for b in range(2):
        pltpu.make_async_copy(src_hbm.at[pl.ds(base + b * CH, CH)],
                              si.at[b], isem[b]).wait()
        pltpu.make_async_copy(dst_hbm.at[pl.ds(base + b * CH, CH)],
                              di.at[b], isem[b]).wait()
        pltpu.async_copy(hs_hbm.at[si.at[b]], rows[b], gsem[b])

    def outer(t, _):
        for j in range(4):
            ci = t * 4 + j
            b = j % 2
            slot = j
            pltpu.make_async_copy(hs_hbm.at[si.at[slot]], rows[b],
                                  gsem[b]).wait()
            pltpu.sync_copy(rows[b], acc.at[di.at[slot]], add=True)
            nl = ci + 4

            @pl.when(nl < KCH)
            def _():
                pltpu.async_copy(src_hbm.at[pl.ds(base + nl * CH, CH)],
                                 si.at[slot], isem[slot])
                pltpu.async_copy(dst_hbm.at[pl.ds(base + nl * CH, CH)],
                                 di.at[slot], isem[slot])

            ng = ci + 2
            slot2 = (j + 2) % 4

            @pl.when(ng < KCH)
            def _():
                pltpu.make_async_copy(
                    src_hbm.at[pl.ds(base + ng * CH, CH)], si.at[slot2],
                    isem[slot2]).wait()
                pltpu.make_async_copy(
                    dst_hbm.at[pl.ds(base + ng * CH, CH)], di.at[slot2],
                    isem[slot2]).wait()
                pltpu.async_copy(hs_hbm.at[si.at[slot2]], rows[b], gsem[b])

        return 0

    lax.fori_loop(0, KCH // 4, outer, 0)
    plsc.subcore_barrier()
    for k in range(ZF):
        pltpu.sync_copy(acc.at[pl.ds(s * RPT + k * CH, CH)],
                        out_hbm.at[pl.ds(c * NP + s * RPT + k * CH, CH)])


_agg_call = pl.kernel(
    _agg_body,
    out_type=jax.ShapeDtypeStruct((2 * NP, D), jnp.float32),
    mesh=_MESH,
    scratch_types=[
        pltpu.VMEM((4, CH), jnp.int32),
        pltpu.VMEM((4, CH), jnp.int32),
        pltpu.VMEM((CH, D), jnp.float32),
        pltpu.VMEM((CH, D), jnp.float32),
        pltpu.VMEM_SHARED((NP, D), jnp.float32),
        pltpu.SemaphoreType.DMA,
        pltpu.SemaphoreType.DMA,
        pltpu.SemaphoreType.DMA,
        pltpu.SemaphoreType.DMA,
        pltpu.SemaphoreType.DMA,
        pltpu.SemaphoreType.DMA,
    ],
)

BR = 1024


def _pre_body(cnt_ref, x_ref, w_ref, dinv_ref, hs_ref):
    deg = cnt_ref[0] + cnt_ref[1] + 1.0
    dinvb = lax.rsqrt(jnp.maximum(deg, 1e-12))
    h = jnp.dot(x_ref[...], w_ref[...], preferred_element_type=jnp.float32)
    dinv_ref[...] = dinvb
    hs_ref[...] = h * dinvb


_pre_call = pl.pallas_call(
    _pre_body,
    grid=(pl.cdiv(N, BR),),
    in_specs=[
        pl.BlockSpec((2, BR, D), lambda i: (0, i, 0)),
        pl.BlockSpec((BR, D), lambda i: (i, 0)),
        pl.BlockSpec((D, D), lambda i: (0, 0)),
    ],
    out_specs=[
        pl.BlockSpec((BR, D), lambda i: (i, 0)),
        pl.BlockSpec((BR, D), lambda i: (i, 0)),
    ],
    out_shape=[
        jax.ShapeDtypeStruct((N, D), jnp.float32),
        jax.ShapeDtypeStruct((N, D), jnp.float32),
    ],
)


def _mid_body(p_ref, hs1_ref, dinv_ref, b1_ref, w2_ref, hs2_ref):
    agg = p_ref[0] + p_ref[1] + hs1_ref[...]
    t = agg * dinv_ref[...] + b1_ref[...]
    t = jnp.maximum(t, 0.0)
    h2 = jnp.dot(t, w2_ref[...], preferred_element_type=jnp.float32)
    hs2_ref[...] = h2 * dinv_ref[...]


_mid_call = pl.pallas_call(
    _mid_body,
    grid=(pl.cdiv(N, BR),),
    in_specs=[
        pl.BlockSpec((2, BR, D), lambda i: (0, i, 0)),
        pl.BlockSpec((BR, D), lambda i: (i, 0)),
        pl.BlockSpec((BR, D), lambda i: (i, 0)),
        pl.BlockSpec((1, D), lambda i: (0, 0)),
        pl.BlockSpec((D, D), lambda i: (0, 0)),
    ],
    out_specs=pl.BlockSpec((BR, D), lambda i: (i, 0)),
    out_shape=jax.ShapeDtypeStruct((N, D), jnp.float32),
)


def _fin_body(q_ref, hs2_ref, dinv_ref, b2_ref, out_ref):
    agg = q_ref[0] + q_ref[1] + hs2_ref[...]
    out_ref[...] = agg * dinv_ref[...] + b2_ref[...]


_fin_call = pl.pallas_call(
    _fin_body,
    grid=(pl.cdiv(N, BR),),
    in_specs=[
        pl.BlockSpec((2, BR, D), lambda i: (0, i, 0)),
        pl.BlockSpec((BR, D), lambda i: (i, 0)),
        pl.BlockSpec((BR, D), lambda i: (i, 0)),
        pl.BlockSpec((1, D), lambda i: (0, 0)),
    ],
    out_specs=pl.BlockSpec((BR, D), lambda i: (i, 0)),
    out_shape=jax.ShapeDtypeStruct((N, D), jnp.float32),
)


def kernel(x, edge_index, W1, b1, W2, b2):
    src = edge_index[0]
    dst = edge_index[1]
    pad = EP - E
    iota = jnp.arange(pad, dtype=src.dtype)
    srcp = jnp.concatenate([src, iota % N])
    dstp = jnp.concatenate([dst, N + iota % (NP - N)])
    cntp = _cnt_call(dstp).reshape(2, NP, D)
    dinvb, hs1 = _pre_call(cntp, x, W1)
    p = _agg_call(hs1, srcp, dstp).reshape(2, NP, D)
    hs2 = _mid_call(p, hs1, dinvb, b1.reshape(1, D), W2)
    q = _agg_call(hs2, srcp, dstp).reshape(2, NP, D)
    return _fin_call(q, hs2, dinvb, b2.reshape(1, D))


# speedup vs baseline: 3.6524x; 1.0866x over previous
import jax
import jax.numpy as jnp
from jax import lax
from jax.experimental import pallas as pl
from jax.experimental.pallas import tpu as pltpu
from jax.experimental.pallas import tpu_sc as plsc

N = 10000
NP = 10240
E = 320000
D = 128
NC = 2
NS = 16
NW = NC * NS
CH = 128
KCH = 80
EPW = CH * KCH
EP = NW * EPW
NB = 4
CHA = 64
KCHA = EPW // CHA
RPT = NP // NS
ZF = RPT // CH

_MESH = plsc.VectorSubcoreMesh(core_axis_name="c", subcore_axis_name="s")


def _fill_rows(buf, nrow, ncol, vec):
    def body(i, _):
        for j in range(ncol // 16):
            buf[i, pl.ds(j * 16, 16)] = vec
        return 0

    lax.fori_loop(0, nrow, body, 0)


def _cnt_body(dst_hbm, out_hbm, di, ones_v, acc, i0, i1):
    c = lax.axis_index("c")
    s = lax.axis_index("s")
    w = c * NS + s
    isem = (i0, i1)
    _fill_rows(ones_v, CH, D, jnp.zeros((16,), jnp.float32))
    for k in range(ZF):
        pltpu.sync_copy(ones_v, acc.at[pl.ds(s * RPT + k * CH, CH)])
    plsc.subcore_barrier()
    _fill_rows(ones_v, CH, D, jnp.ones((16,), jnp.float32))
    base = w * EPW
    for j in range(2):
        pltpu.async_copy(dst_hbm.at[pl.ds(base + j * CH, CH)], di.at[j],
                         isem[j])

    def chunk(t, _):
        for slot in range(2):
            ci = t * 2 + slot
            pltpu.make_async_copy(dst_hbm.at[pl.ds(base + ci * CH, CH)],
                                  di.at[slot], isem[slot]).wait()
            pltpu.sync_copy(ones_v, acc.at[di.at[slot]], add=True)
            nl = ci + 2

            @pl.when(nl < KCH)
            def _():
                pltpu.async_copy(dst_hbm.at[pl.ds(base + nl * CH, CH)],
                                 di.at[slot], isem[slot])

        return 0

    lax.fori_loop(0, KCH // 2, chunk, 0)
    plsc.subcore_barrier()
    for k in range(ZF):
        pltpu.sync_copy(acc.at[pl.ds(s * RPT + k * CH, CH)],
                        out_hbm.at[pl.ds(c * NP + s * RPT + k * CH, CH)])


_cnt_call = pl.kernel(
    _cnt_body,
    out_type=jax.ShapeDtypeStruct((2 * NP, D), jnp.float32),
    mesh=_MESH,
    scratch_types=[
        pltpu.VMEM((2, CH), jnp.int32),
        pltpu.VMEM((CH, D), jnp.float32),
        pltpu.VMEM_SHARED((NP, D), jnp.float32),
        pltpu.SemaphoreType.DMA,
        pltpu.SemaphoreType.DMA,
    ],
)


def _agg_body(hs_hbm, src_hbm, dst_hbm, out_hbm,
              si, di, r0, r1, r2, r3, acc,
              i0, i1, i2, i3, i4, i5, i6, i7, g0, g1, g2, g3):
    c = lax.axis_index("c")
    s = lax.axis_index("s")
    w = c * NS + s
    rows = (r0, r1, r2, r3)
    gsem = (g0, g1, g2, g3)
    isem = (i0, i1, i2, i3, i4, i5, i6, i7)
    _fill_rows(r0, CHA, D, jnp.zeros((16,), jnp.float32))
    for k in range(RPT // CHA):
        pltpu.sync_copy(r0, acc.at[pl.ds(s * RPT + k * CHA, CHA)])
    plsc.subcore_barrier()
    base = w * EPW
    for j in range(8):
        pltpu.async_copy(src_hbm.at[pl.ds(base + j * CHA, CHA)], si.at[j],
                         isem[j])
        pltpu.async_copy(dst_hbm.at[pl.ds(base + j * CHA, CHA)], di.at[j],
                         isem[j])
    for b in range(4):
        pltpu.make_async_copy(src_hbm.at[pl.ds(base + b * CHA, CHA)],
                              si.at[b], isem[b]).wait()
        pltpu.make_async_copy(dst_hbm.at[pl.ds(base + b * CHA, CHA)],
                              di.at[b], isem[b]).wait()
        pltpu.async_copy(hs_hbm.at[si.at[b]], rows[b], gsem[b])

    def outer(t, _):
        for j in range(8):
            ci = t * 8 + j
            b = j % 4
            slot = j
            pltpu.make_async_copy(hs_hbm.at[si.at[slot]], rows[b],
                                  gsem[b]).wait()
            pltpu.sync_copy(rows[b], acc.at[di.at[slot]], add=True)
            nl = ci + 8

            @pl.when(nl < KCHA)
            def _():
                pltpu.async_copy(src_hbm.at[pl.ds(base + nl * CHA, CHA)],
                                 si.at[slot], isem[slot])
                pltpu.async_copy(dst_hbm.at[pl.ds(base + nl * CHA, CHA)],
                                 di.at[slot], isem[slot])

            ng = ci + 4
            slot2 = (j + 4) % 8

            @pl.when(ng < KCHA)
            def _():
                pltpu.make_async_copy(
                    src_hbm.at[pl.ds(base + ng * CHA, CHA)], si.at[slot2],
                    isem[slot2]).wait()
                pltpu.make_async_copy(
                    dst_hbm.at[pl.ds(base + ng * CHA, CHA)], di.at[slot2],
                    isem[slot2]).wait()
                pltpu.async_copy(hs_hbm.at[si.at[slot2]], rows[b], gsem[b])

        return 0

    lax.fori_loop(0, KCHA // 8, outer, 0)
    plsc.subcore_barrier()
    for k in range(ZF):
        pltpu.sync_copy(acc.at[pl.ds(s * RPT + k * CH, CH)],
                        out_hbm.at[pl.ds(c * NP + s * RPT + k * CH, CH)])


_agg_call = pl.kernel(
    _agg_body,
    out_type=jax.ShapeDtypeStruct((2 * NP, D), jnp.float32),
    mesh=_MESH,
    scratch_types=[
        pltpu.VMEM((8, CHA), jnp.int32),
        pltpu.VMEM((8, CHA), jnp.int32),
        pltpu.VMEM((CHA, D), jnp.float32),
        pltpu.VMEM((CHA, D), jnp.float32),
        pltpu.VMEM((CHA, D), jnp.float32),
        pltpu.VMEM((CHA, D), jnp.float32),
        pltpu.VMEM_SHARED((NP, D), jnp.float32),
    ] + [pltpu.SemaphoreType.DMA] * 12,
)

BR = 1024


def _pre_body(cnt_ref, x_ref, w_ref, dinv_ref, hs_ref):
    deg = cnt_ref[0] + cnt_ref[1] + 1.0
    dinvb = lax.rsqrt(jnp.maximum(deg, 1e-12))
    h = jnp.dot(x_ref[...], w_ref[...], preferred_element_type=jnp.float32)
    dinv_ref[...] = dinvb
    hs_ref[...] = h * dinvb


_pre_call = pl.pallas_call(
    _pre_body,
    grid=(pl.cdiv(N, BR),),
    in_specs=[
        pl.BlockSpec((2, BR, D), lambda i: (0, i, 0)),
        pl.BlockSpec((BR, D), lambda i: (i, 0)),
        pl.BlockSpec((D, D), lambda i: (0, 0)),
    ],
    out_specs=[
        pl.BlockSpec((BR, D), lambda i: (i, 0)),
        pl.BlockSpec((BR, D), lambda i: (i, 0)),
    ],
    out_shape=[
        jax.ShapeDtypeStruct((N, D), jnp.float32),
        jax.ShapeDtypeStruct((N, D), jnp.float32),
    ],
)


def _mid_body(p_ref, hs1_ref, dinv_ref, b1_ref, w2_ref, hs2_ref):
    agg = p_ref[0] + p_ref[1] + hs1_ref[...]
    t = agg * dinv_ref[...] + b1_ref[...]
    t = jnp.maximum(t, 0.0)
    h2 = jnp.dot(t, w2_ref[...], preferred_element_type=jnp.float32)
    hs2_ref[...] = h2 * dinv_ref[...]


_mid_call = pl.pallas_call(
    _mid_body,
    grid=(pl.cdiv(N, BR),),
    in_specs=[
        pl.BlockSpec((2, BR, D), lambda i: (0, i, 0)),
        pl.BlockSpec((BR, D), lambda i: (i, 0)),
        pl.BlockSpec((BR, D), lambda i: (i, 0)),
        pl.BlockSpec((1, D), lambda i: (0, 0)),
        pl.BlockSpec((D, D), lambda i: (0, 0)),
    ],
    out_specs=pl.BlockSpec((BR, D), lambda i: (i, 0)),
    out_shape=jax.ShapeDtypeStruct((N, D), jnp.float32),
)


def _fin_body(q_ref, hs2_ref, dinv_ref, b2_ref, out_ref):
    agg = q_ref[0] + q_ref[1] + hs2_ref[...]
    out_ref[...] = agg * dinv_ref[...] + b2_ref[...]


_fin_call = pl.pallas_call(
    _fin_body,
    grid=(pl.cdiv(N, BR),),
    in_specs=[
        pl.BlockSpec((2, BR, D), lambda i: (0, i, 0)),
        pl.BlockSpec((BR, D), lambda i: (i, 0)),
        pl.BlockSpec((BR, D), lambda i: (i, 0)),
        pl.BlockSpec((1, D), lambda i: (0, 0)),
    ],
    out_specs=pl.BlockSpec((BR, D), lambda i: (i, 0)),
    out_shape=jax.ShapeDtypeStruct((N, D), jnp.float32),
)


def kernel(x, edge_index, W1, b1, W2, b2):
    src = edge_index[0]
    dst = edge_index[1]
    pad = EP - E
    iota = jnp.arange(pad, dtype=src.dtype)
    srcp = jnp.concatenate([src, iota % N])
    dstp = jnp.concatenate([dst, N + iota % (NP - N)])
    cntp = _cnt_call(dstp).reshape(2, NP, D)
    dinvb, hs1 = _pre_call(cntp, x, W1)
    p = _agg_call(hs1, srcp, dstp).reshape(2, NP, D)
    hs2 = _mid_call(p, hs1, dinvb, b1.reshape(1, D), W2)
    q = _agg_call(hs2, srcp, dstp).reshape(2, NP, D)
    return _fin_call(q, hs2, dinvb, b2.reshape(1, D))
